# baseline (speedup 1.0000x reference)
import jax
import jax.numpy as jnp
from jax import lax
from jax.experimental import pallas as pl
from jax.experimental.pallas import tpu as pltpu
from jax.experimental.pallas import tpu_sc as plsc

SEQ = 200
BATCH = 4096
EMB = 32
VOCAB = 1000000
NC = 2
NS = 16
NW = NC * NS
BPW = BATCH // NW
CH = 4
NSLOT = 2 * CH

NBLK = VOCAB // 128
TAILP = (VOCAB - 64) * EMB // 128


def _transpose_body(tblt_hbm, tail_hbm, out_hbm,
                    tiles_v, outs_v, tail_v, sem_a, sem_b, sem_oa, sem_ob):
  wid = lax.axis_index("s") * NC + lax.axis_index("c")

  @pl.when(wid == NW - 1)
  def _():
    pltpu.sync_copy(tail_hbm, tail_v)
    pltpu.sync_copy(tail_v, out_hbm.at[pl.ds(TAILP, 16), :])

  lanes = jnp.arange(16, dtype=jnp.int32)
  rots = [(lanes + k) % 16 for k in range(16)]
  dvs = [16 * h + lanes for h in range(2)]
  wrow = [[(32 * rots[k] + 16 * h + lanes) // 128 for k in range(16)]
          for h in range(2)]
  wcol = [[(32 * rots[k] + 16 * h + lanes) % 128 for k in range(16)]
          for h in range(2)]

  def blk(j):
    return wid + NW * j

  def start_blk(c, slot, sem):
    @pl.when(c < NBLK)
    def _():
      pltpu.async_copy(
          tblt_hbm.at[:, pl.ds(c * 128, 128)], tiles_v.at[slot], sem)

  def wait_blk(c, slot, sem):
    @pl.when(c < NBLK)
    def _():
      pltpu.make_async_copy(
          tblt_hbm.at[:, pl.ds(0, 128)], tiles_v.at[slot], sem).wait()

  def wait_out(c, slot, sem):
    @pl.when(c < NBLK)
    def _():
      pltpu.make_async_copy(
          outs_v.at[slot], out_hbm.at[pl.ds(0, 32), :], sem).wait()

  def process_blk(c, slot, sem):
    @pl.when(c < NBLK)
    def _():
      tile = tiles_v.at[slot]
      outs = outs_v.at[slot]

      @plsc.parallel_loop(0, 8, 1, unroll=2)
      def _(g):
        g16 = g * 16
        g4 = g * 4
        for h in range(2):
          for k in range(16):
            x = plsc.load_gather(tile, [dvs[h], g16 + rots[k]])
            plsc.store_scatter(outs, [g4 + wrow[h][k], wcol[h][k]], x)

      pltpu.async_copy(
          outs_v.at[slot], out_hbm.at[pl.ds(c * 32, 32), :], sem)

  def start4(j0, s0, sem):
    for k in range(4):
      start_blk(blk(j0 + k), s0 + k, sem)

  def wait4(j0, s0, sem):
    for k in range(4):
      wait_blk(blk(j0 + k), s0 + k, sem)

  def waitout4(j0, s0, sem):
    for k in range(4):
      wait_out(blk(j0 + k), s0 + k, sem)

  def proc4(j0, s0, sem):
    for k in range(4):
      process_blk(blk(j0 + k), s0 + k, sem)

  start4(0, 0, sem_a)
  start4(4, 4, sem_b)

  def body(i, _):
    j0 = 8 * i
    wait4(j0, 0, sem_a)

    @pl.when(i >= 1)
    def _():
      waitout4(j0 - 8, 0, sem_oa)

    proc4(j0, 0, sem_oa)
    start4(j0 + 8, 0, sem_a)
    wait4(j0 + 4, 4, sem_b)

    @pl.when(i >= 1)
    def _():
      waitout4(j0 - 4, 4, sem_ob)

    proc4(j0 + 4, 4, sem_ob)
    start4(j0 + 12, 4, sem_b)
    return 0

  lax.fori_loop(0, 31, body, 0)
  waitout4(240, 0, sem_oa)
  waitout4(244, 4, sem_ob)


def _danmodel_body(inp_hbm, tbl_hbm, wb_hbm, out_hbm,
                   idx_v, rows_v, acc_v, wb_v, out_v, sem_a, sem_b):
  wid = lax.axis_index("s") * NC + lax.axis_index("c")
  base = wid * BPW

  pltpu.sync_copy(inp_hbm.at[:, pl.ds(base, BPW)], idx_v)
  pltpu.sync_copy(wb_hbm, wb_v)

  zeros = jnp.zeros((16,), jnp.float32)

  @plsc.parallel_loop(0, BPW, 1, unroll=4)
  def _(b):
    acc_v[b, pl.ds(0, 16)] = zeros
    acc_v[b, pl.ds(16, 16)] = zeros

  def start_group(s0, c0, sem):
    for k in range(CH):
      pltpu.async_copy(
          tbl_hbm.at[idx_v.at[s0 + k]],
          rows_v.at[pl.ds((c0 + k) * BPW, BPW), :],
          sem)

  def drain_group(c0, sem):
    for k in range(CH):
      pltpu.make_async_copy(
          tbl_hbm.at[idx_v.at[0]],
          rows_v.at[pl.ds((c0 + k) * BPW, BPW), :],
          sem).wait()

  def accum_group(c0):
    @plsc.parallel_loop(0, BPW, 1, unroll=2)
    def _(b):
      for h in range(2):
        sl = pl.ds(h * 16, 16)
        v01 = rows_v[(c0 + 0) * BPW + b, sl] + rows_v[(c0 + 1) * BPW + b, sl]
        v23 = rows_v[(c0 + 2) * BPW + b, sl] + rows_v[(c0 + 3) * BPW + b, sl]
        plsc.addupdate(acc_v.at[b, sl], v01 + v23)

  start_group(0, 0, sem_a)

  def body(i, _):
    s0 = (2 * CH) * i
    drain_group(0, sem_a)
    start_group(s0 + CH, CH, sem_b)
    accum_group(0)
    drain_group(CH, sem_b)

    @pl.when(s0 + 2 * CH < SEQ)
    def _():
      start_group(s0 + 2 * CH, 0, sem_a)

    accum_group(CH)
    return 0

  lax.fori_loop(0, SEQ // (2 * CH), body, 0)

  inv_seq = jnp.float32(1.0 / SEQ)
  w_lo = wb_v[pl.ds(0, 16)]
  w_hi = wb_v[pl.ds(16, 16)]
  bias = wb_v[pl.ds(EMB, 16)][0]
  lanes = jnp.arange(16, dtype=jnp.int32)

  def ep_body(g, _):
    idxb = g * 16 + lanes
    o = jnp.full((16,), bias, jnp.float32)
    for d in range(EMB):
      col = plsc.load_gather(acc_v, [idxb, jnp.full((16,), d, jnp.int32)])
      x = col * inv_seq
      e = jnp.exp(jnp.abs(x) * 2.0)
      t = jnp.sign(x) * (1.0 - 2.0 / (e + 1.0))
      w_d = w_lo[d] if d < 16 else w_hi[d - 16]
      o = o + t * w_d
    out_v[pl.ds(g * 16, 16)] = o
    return 0

  lax.fori_loop(0, BPW // 16, ep_body, 0)

  pltpu.sync_copy(out_v, out_hbm.at[pl.ds(base, BPW)])


@jax.jit
def _run(inp, tblt, tail, wb):
  mesh = plsc.VectorSubcoreMesh(
      core_axis_name="c", subcore_axis_name="s", num_cores=NC,
      num_subcores=NS)
  tlin = pl.kernel(
      _transpose_body,
      out_type=jax.ShapeDtypeStruct((VOCAB * EMB // 128, 128), jnp.float32),
      mesh=mesh,
      compiler_params=pltpu.CompilerParams(
          needs_layout_passes=False, use_tc_tiling_on_sc=True),
      scratch_types=[
          pltpu.VMEM((8, 32, 128), jnp.float32),
          pltpu.VMEM((8, 32, 128), jnp.float32),
          pltpu.VMEM((16, 128), jnp.float32),
          pltpu.SemaphoreType.DMA,
          pltpu.SemaphoreType.DMA,
          pltpu.SemaphoreType.DMA,
          pltpu.SemaphoreType.DMA,
      ],
  )(tblt, tail)
  tbl = tlin.reshape(VOCAB, EMB)
  return pl.kernel(
      _danmodel_body,
      out_type=jax.ShapeDtypeStruct((BATCH,), jnp.float32),
      mesh=mesh,
      compiler_params=pltpu.CompilerParams(
          needs_layout_passes=False, use_tc_tiling_on_sc=False),
      scratch_types=[
          pltpu.VMEM((SEQ, BPW), jnp.int32),
          pltpu.VMEM((NSLOT * BPW, EMB), jnp.float32),
          pltpu.VMEM((BPW, EMB), jnp.float32),
          pltpu.VMEM((64,), jnp.float32),
          pltpu.VMEM((BPW,), jnp.float32),
          pltpu.SemaphoreType.DMA,
          pltpu.SemaphoreType.DMA,
      ],
  )(inp, tbl, wb)


def kernel(input, emb_table, W, b):
  inp = input.astype(jnp.int32)
  tblt = emb_table.T
  tail = emb_table[VOCAB - 64:].reshape(16, 128)
  wb = jnp.concatenate(
      [W.reshape(-1), b.reshape(-1),
       jnp.zeros((64 - EMB - 1,), jnp.float32)]).astype(jnp.float32)
  out = _run(inp, tblt, tail, wb)
  return out.reshape(BATCH, 1)


# speedup vs baseline: 1.0977x; 1.0977x over previous
import jax
import jax.numpy as jnp
from jax import lax
from jax.experimental import pallas as pl
from jax.experimental.pallas import tpu as pltpu
from jax.experimental.pallas import tpu_sc as plsc

SEQ = 200
BATCH = 4096
EMB = 32
VOCAB = 1000000
NC = 2
NS = 16
NW = NC * NS
BPW = BATCH // NW
CH = 4
NSLOT = 2 * CH

NBLK = VOCAB // 128
TAILP = (VOCAB - 64) * EMB // 128


def _transpose_body(tblt_hbm, tail_hbm, out_hbm,
                    tiles_v, outs_v, tail_v, sem_a, sem_b, sem_oa, sem_ob):
  wid = lax.axis_index("s") * NC + lax.axis_index("c")

  @pl.when(wid == NW - 1)
  def _():
    pltpu.sync_copy(tail_hbm, tail_v)
    pltpu.sync_copy(tail_v, out_hbm.at[pl.ds(TAILP, 16), :])

  lanes = jnp.arange(16, dtype=jnp.int32)
  rots = [(lanes + k) % 16 for k in range(16)]
  dvs = [16 * h + lanes for h in range(2)]
  wrow = [[(32 * rots[k] + 16 * h + lanes) // 128 for k in range(16)]
          for h in range(2)]
  wcol = [[(32 * rots[k] + 16 * h + lanes) % 128 for k in range(16)]
          for h in range(2)]

  def blk(j):
    return wid + NW * j

  def start_blk(c, slot, sem):
    @pl.when(c < NBLK)
    def _():
      pltpu.async_copy(
          tblt_hbm.at[:, pl.ds(c * 128, 128)], tiles_v.at[slot], sem)

  def wait_blk(c, slot, sem):
    @pl.when(c < NBLK)
    def _():
      pltpu.make_async_copy(
          tblt_hbm.at[:, pl.ds(0, 128)], tiles_v.at[slot], sem).wait()

  def wait_out(c, slot, sem):
    @pl.when(c < NBLK)
    def _():
      pltpu.make_async_copy(
          outs_v.at[slot], out_hbm.at[pl.ds(0, 32), :], sem).wait()

  def process_blk(c, slot, sem):
    @pl.when(c < NBLK)
    def _():
      tile = tiles_v.at[slot]
      outs = outs_v.at[slot]

      @plsc.parallel_loop(0, 8, 1, unroll=1)
      def _(g):
        g16 = g * 16
        g4 = g * 4
        for h in range(2):
          for k in range(16):
            x = plsc.load_gather(tile, [dvs[h], g16 + rots[k]])
            plsc.store_scatter(outs, [g4 + wrow[h][k], wcol[h][k]], x)

      pltpu.async_copy(
          outs_v.at[slot], out_hbm.at[pl.ds(c * 32, 32), :], sem)

  def start4(j0, s0, sem):
    for k in range(4):
      start_blk(blk(j0 + k), s0 + k, sem)

  def wait4(j0, s0, sem):
    for k in range(4):
      wait_blk(blk(j0 + k), s0 + k, sem)

  def waitout4(j0, s0, sem):
    for k in range(4):
      wait_out(blk(j0 + k), s0 + k, sem)

  def proc4(j0, s0, sem):
    for k in range(4):
      process_blk(blk(j0 + k), s0 + k, sem)

  start4(0, 0, sem_a)
  start4(4, 4, sem_b)

  def body(i, _):
    j0 = 8 * i
    wait4(j0, 0, sem_a)

    @pl.when(i >= 1)
    def _():
      waitout4(j0 - 8, 0, sem_oa)

    proc4(j0, 0, sem_oa)
    start4(j0 + 8, 0, sem_a)
    wait4(j0 + 4, 4, sem_b)

    @pl.when(i >= 1)
    def _():
      waitout4(j0 - 4, 4, sem_ob)

    proc4(j0 + 4, 4, sem_ob)
    start4(j0 + 12, 4, sem_b)
    return 0

  lax.fori_loop(0, 31, body, 0)
  waitout4(240, 0, sem_oa)
  waitout4(244, 4, sem_ob)


def _danmodel_body(inp_hbm, tbl_hbm, wb_hbm, out_hbm,
                   idx_v, rows_v, acc_v, wb_v, out_v, sem_a, sem_b):
  wid = lax.axis_index("s") * NC + lax.axis_index("c")
  base = wid * BPW

  pltpu.sync_copy(inp_hbm.at[:, pl.ds(base, BPW)], idx_v)
  pltpu.sync_copy(wb_hbm, wb_v)

  zeros = jnp.zeros((16,), jnp.float32)

  @plsc.parallel_loop(0, BPW, 1, unroll=4)
  def _(b):
    acc_v[b, pl.ds(0, 16)] = zeros
    acc_v[b, pl.ds(16, 16)] = zeros

  def start_group(s0, c0, sem):
    for k in range(CH):
      pltpu.async_copy(
          tbl_hbm.at[idx_v.at[s0 + k]],
          rows_v.at[pl.ds((c0 + k) * BPW, BPW), :],
          sem)

  def drain_group(c0, sem):
    for k in range(CH):
      pltpu.make_async_copy(
          tbl_hbm.at[idx_v.at[0]],
          rows_v.at[pl.ds((c0 + k) * BPW, BPW), :],
          sem).wait()

  def accum_group(c0):
    @plsc.parallel_loop(0, BPW, 1, unroll=2)
    def _(b):
      for h in range(2):
        sl = pl.ds(h * 16, 16)
        v01 = rows_v[(c0 + 0) * BPW + b, sl] + rows_v[(c0 + 1) * BPW + b, sl]
        v23 = rows_v[(c0 + 2) * BPW + b, sl] + rows_v[(c0 + 3) * BPW + b, sl]
        plsc.addupdate(acc_v.at[b, sl], v01 + v23)

  start_group(0, 0, sem_a)

  def body(i, _):
    s0 = (2 * CH) * i
    drain_group(0, sem_a)
    start_group(s0 + CH, CH, sem_b)
    accum_group(0)
    drain_group(CH, sem_b)

    @pl.when(s0 + 2 * CH < SEQ)
    def _():
      start_group(s0 + 2 * CH, 0, sem_a)

    accum_group(CH)
    return 0

  lax.fori_loop(0, SEQ // (2 * CH), body, 0)

  inv_seq = jnp.float32(1.0 / SEQ)
  w_lo = wb_v[pl.ds(0, 16)]
  w_hi = wb_v[pl.ds(16, 16)]
  bias = wb_v[pl.ds(EMB, 16)][0]
  lanes = jnp.arange(16, dtype=jnp.int32)

  def ep_body(g, _):
    idxb = g * 16 + lanes
    o = jnp.full((16,), bias, jnp.float32)
    for d in range(EMB):
      col = plsc.load_gather(acc_v, [idxb, jnp.full((16,), d, jnp.int32)])
      x = col * inv_seq
      e = jnp.exp(jnp.abs(x) * 2.0)
      t = jnp.sign(x) * (1.0 - 2.0 / (e + 1.0))
      w_d = w_lo[d] if d < 16 else w_hi[d - 16]
      o = o + t * w_d
    out_v[pl.ds(g * 16, 16)] = o
    return 0

  lax.fori_loop(0, BPW // 16, ep_body, 0)

  pltpu.sync_copy(out_v, out_hbm.at[pl.ds(base, BPW)])


@jax.jit
def _run(inp, tblt, tail, wb):
  mesh = plsc.VectorSubcoreMesh(
      core_axis_name="c", subcore_axis_name="s", num_cores=NC,
      num_subcores=NS)
  tlin = pl.kernel(
      _transpose_body,
      out_type=jax.ShapeDtypeStruct((VOCAB * EMB // 128, 128), jnp.float32),
      mesh=mesh,
      compiler_params=pltpu.CompilerParams(
          needs_layout_passes=False, use_tc_tiling_on_sc=True),
      scratch_types=[
          pltpu.VMEM((8, 32, 128), jnp.float32),
          pltpu.VMEM((8, 32, 128), jnp.float32),
          pltpu.VMEM((16, 128), jnp.float32),
          pltpu.SemaphoreType.DMA,
          pltpu.SemaphoreType.DMA,
          pltpu.SemaphoreType.DMA,
          pltpu.SemaphoreType.DMA,
      ],
  )(tblt, tail)
  tbl = tlin.reshape(VOCAB, EMB)
  return pl.kernel(
      _danmodel_body,
      out_type=jax.ShapeDtypeStruct((BATCH,), jnp.float32),
      mesh=mesh,
      compiler_params=pltpu.CompilerParams(
          needs_layout_passes=False, use_tc_tiling_on_sc=False),
      scratch_types=[
          pltpu.VMEM((SEQ, BPW), jnp.int32),
          pltpu.VMEM((NSLOT * BPW, EMB), jnp.float32),
          pltpu.VMEM((BPW, EMB), jnp.float32),
          pltpu.VMEM((64,), jnp.float32),
          pltpu.VMEM((BPW,), jnp.float32),
          pltpu.SemaphoreType.DMA,
          pltpu.SemaphoreType.DMA,
      ],
  )(inp, tbl, wb)


def kernel(input, emb_table, W, b):
  inp = input.astype(jnp.int32)
  tblt = emb_table.T
  tail = emb_table[VOCAB - 64:].reshape(16, 128)
  wb = jnp.concatenate(
      [W.reshape(-1), b.reshape(-1),
       jnp.zeros((64 - EMB - 1,), jnp.float32)]).astype(jnp.float32)
  out = _run(inp, tblt, tail, wb)
  return out.reshape(BATCH, 1)
